# baseline (speedup 1.0000x reference)
import jax
import jax.numpy as jnp
from jax import lax
from jax.experimental import pallas as pl
from jax.experimental.pallas import tpu as pltpu
from jax.experimental.pallas import tpu_sc as plsc

B = 4096
L = 3335
LP = 3456
HL = LP // 2
NW = 32
BPW = B // NW
VOCAB = 17346050
VS4 = (VOCAB + 3) // 4
PAD4 = VS4 * 4 - VOCAB
NC16 = HL // 16


def _sc_body(sidx_hbm, ridx_hbm, tab_hbm, w_hbm, dw_hbm, out_hbm,
             sbuf0, sbuf1, ibuf0, ibuf1, rb0, rb1,
             w0_v, w1_v, dw_v, acc0_v, acc1_v, out_v,
             sem_g, sem_i):
  cid = lax.axis_index("c")
  sid = lax.axis_index("s")
  wid = sid * 2 + cid
  b0 = wid * BPW
  h0 = b0 * 2

  pltpu.sync_copy(w_hbm.at[0], w0_v)
  pltpu.sync_copy(w_hbm.at[1], w1_v)
  pltpu.sync_copy(dw_hbm, dw_v)

  iota = lax.iota(jnp.int32, 16)
  zeros16 = iota * 0
  lane0 = iota == 0
  zerof = jnp.zeros((16,), jnp.float32)

  sbufs = (sbuf0, sbuf1)
  ibufs = (ibuf0, ibuf1)
  rbufs = (rb0, rb1)

  def gathers(p, issue):
    c = pltpu.make_async_copy(tab_hbm.at[sbufs[p]], rbufs[p], sem_g)
    if issue:
      c.start()
    return (c,)

  def idx_dma(p, h, issue):
    c1 = pltpu.make_async_copy(sidx_hbm.at[h0 + h], sbufs[p], sem_i)
    c2 = pltpu.make_async_copy(ridx_hbm.at[h0 + h], ibufs[p], sem_i)
    if issue:
      c1.start()
      c2.start()
    return c1, c2

  def compute_half(p, carry):
    ib = ibufs[p]
    rb = rbufs[p]
    woff = p * HL

    def chunk_body(m, carry):
      a0, a1 = carry
      base = m * 16
      iv = ib[pl.ds(base, 16)]
      c = lax.shift_left(lax.bitwise_and(iv, 3), 1)
      r = base + iota
      r0 = plsc.load_gather(rb, [r, c])
      r1 = plsc.load_gather(rb, [r, c + 1])
      w0c = w0_v[pl.ds(woff + base, 16)]
      w1c = w1_v[pl.ds(woff + base, 16)]
      return (a0 + r0 * w0c, a1 + r1 * w1c)

    return lax.fori_loop(0, NC16, chunk_body, carry, unroll=8)

  for c in idx_dma(0, 0, True):
    c.wait()
  gathers(0, True)
  idx_dma(1, 1, True)

  def batch_body(i, _):
    carry = (zerof, zerof)
    for p in (0, 1):
      h = 2 * i + p
      for c in gathers(p, False):
        c.wait()
      @pl.when(h + 1 < 2 * BPW)
      def _():
        for c in idx_dma(1 - p, h + 1, False):
          c.wait()
        gathers(1 - p, True)
      carry = compute_half(p, carry)
      @pl.when(h + 2 < 2 * BPW)
      def _():
        idx_dma(p, h + 2, True)
    a0, a1 = carry
    ivec = zeros16 + i
    plsc.store_scatter(acc0_v, [ivec], zerof + jnp.sum(a0), mask=lane0)
    plsc.store_scatter(acc1_v, [ivec], zerof + jnp.sum(a1), mask=lane0)
    return 0

  lax.fori_loop(0, BPW, batch_body, 0)

  dwv = dw_v[...]
  dw0 = dwv[0]
  dw1 = dwv[1]
  for t in range(BPW // 16):
    a0 = acc0_v[pl.ds(t * 16, 16)]
    a1 = acc1_v[pl.ds(t * 16, 16)]
    h0v = a0 * jnp.clip(a0 + 3.0, 0.0, 6.0) * (1.0 / 6.0)
    h1v = a1 * jnp.clip(a1 + 3.0, 0.0, 6.0) * (1.0 / 6.0)
    y = h0v * dw0 + h1v * dw1
    z = jnp.exp(-2.0 * jnp.abs(y))
    out_v[pl.ds(t * 16, 16)] = jnp.sign(y) * (1.0 - z) / (1.0 + z)

  pltpu.sync_copy(out_v, out_hbm.at[pl.ds(b0, BPW)])


@jax.jit
def kernel(inputs, table, conv_w, dense_w):
  idx = jnp.pad(inputs.astype(jnp.int32), ((0, 0), (0, LP - L)))
  sidx = lax.shift_right_logical(idx, 2).reshape(2 * B, HL)
  ridx = idx.reshape(2 * B, HL)
  p0 = jnp.pad(table[:, 0], (0, PAD4)).reshape(VS4, 4)
  p1 = jnp.pad(table[:, 1], (0, PAD4)).reshape(VS4, 4)
  tab8 = jnp.concatenate([p0, p1], axis=1)[:, jnp.array([0, 4, 1, 5, 2, 6, 3, 7])]
  w2 = jnp.pad(conv_w[:, 0, :].astype(jnp.float32), ((0, 0), (0, LP - L)))
  dw = jnp.pad(dense_w.reshape(2).astype(jnp.float32), (0, 14))

  mesh = plsc.VectorSubcoreMesh(core_axis_name="c", subcore_axis_name="s")
  out = pl.kernel(
      _sc_body,
      out_type=jax.ShapeDtypeStruct((B,), jnp.float32),
      mesh=mesh,
      compiler_params=pltpu.CompilerParams(
          needs_layout_passes=False, use_tc_tiling_on_sc=False),
      scratch_types=[
          pltpu.VMEM((HL,), jnp.int32),
          pltpu.VMEM((HL,), jnp.int32),
          pltpu.VMEM((HL,), jnp.int32),
          pltpu.VMEM((HL,), jnp.int32),
          pltpu.VMEM((HL, 8), jnp.float32),
          pltpu.VMEM((HL, 8), jnp.float32),
          pltpu.VMEM((LP,), jnp.float32),
          pltpu.VMEM((LP,), jnp.float32),
          pltpu.VMEM((16,), jnp.float32),
          pltpu.VMEM((BPW,), jnp.float32),
          pltpu.VMEM((BPW,), jnp.float32),
          pltpu.VMEM((BPW,), jnp.float32),
          pltpu.SemaphoreType.DMA,
          pltpu.SemaphoreType.DMA,
      ],
  )(sidx, ridx, tab8, w2, dw)
  return out.reshape(B, 1)


# speedup vs baseline: 2.6938x; 2.6938x over previous
import jax
import jax.numpy as jnp
from jax import lax
from jax.experimental import pallas as pl
from jax.experimental.pallas import tpu as pltpu
from jax.experimental.pallas import tpu_sc as plsc

B = 4096
L = 3335
LP = 3456
HL = LP // 2
NW = 32
BPW = B // NW
VOCAB = 17346050
VSP = (VOCAB + 7) // 8
PADP = VSP * 8 - VOCAB
NC16 = HL // 16


def _sc_body(sidx_hbm, ridx_hbm, tab0_hbm, tab1_hbm, w_hbm, dw_hbm, out_hbm,
             sbuf0, sbuf1, ibuf0, ibuf1, r0b0, r0b1, r1b0, r1b1,
             w0_v, w1_v, dw_v, acc0_v, acc1_v, out_v,
             sem_g, sem_i):
  cid = lax.axis_index("c")
  sid = lax.axis_index("s")
  wid = sid * 2 + cid
  b0 = wid * BPW
  h0 = b0 * 2

  pltpu.sync_copy(w_hbm.at[0], w0_v)
  pltpu.sync_copy(w_hbm.at[1], w1_v)
  pltpu.sync_copy(dw_hbm, dw_v)

  iota = lax.iota(jnp.int32, 16)
  zeros16 = iota * 0
  lane0 = iota == 0
  zerof = jnp.zeros((16,), jnp.float32)

  sbufs = (sbuf0, sbuf1)
  ibufs = (ibuf0, ibuf1)
  r0bufs = (r0b0, r0b1)
  r1bufs = (r1b0, r1b1)

  def gathers(p, issue):
    c1 = pltpu.make_async_copy(tab0_hbm.at[sbufs[p]], r0bufs[p], sem_g)
    c2 = pltpu.make_async_copy(tab1_hbm.at[sbufs[p]], r1bufs[p], sem_g)
    if issue:
      c1.start()
      c2.start()
    return c1, c2

  def idx_dma(p, h, issue):
    c1 = pltpu.make_async_copy(sidx_hbm.at[h0 + h], sbufs[p], sem_i)
    c2 = pltpu.make_async_copy(ridx_hbm.at[h0 + h], ibufs[p], sem_i)
    if issue:
      c1.start()
      c2.start()
    return c1, c2

  def compute_half(p, carry):
    ib = ibufs[p]
    r0b = r0bufs[p]
    r1b = r1bufs[p]
    woff = p * HL

    def chunk_body(m, carry):
      a0, a1 = carry
      base = m * 16
      iv = ib[pl.ds(base, 16)]
      c = lax.bitwise_and(iv, 7)
      r = base + iota
      r0 = plsc.load_gather(r0b, [r, c])
      r1 = plsc.load_gather(r1b, [r, c])
      w0c = w0_v[pl.ds(woff + base, 16)]
      w1c = w1_v[pl.ds(woff + base, 16)]
      return (a0 + r0 * w0c, a1 + r1 * w1c)

    return lax.fori_loop(0, NC16, chunk_body, carry, unroll=8)

  for c in idx_dma(0, 0, True):
    c.wait()
  gathers(0, True)
  idx_dma(1, 1, True)

  def batch_body(i, _):
    carry = (zerof, zerof)
    for p in (0, 1):
      h = 2 * i + p
      for c in gathers(p, False):
        c.wait()
      @pl.when(h + 1 < 2 * BPW)
      def _():
        for c in idx_dma(1 - p, h + 1, False):
          c.wait()
        gathers(1 - p, True)
      carry = compute_half(p, carry)
      @pl.when(h + 2 < 2 * BPW)
      def _():
        idx_dma(p, h + 2, True)
    a0, a1 = carry
    ivec = zeros16 + i
    plsc.store_scatter(acc0_v, [ivec], zerof + jnp.sum(a0), mask=lane0)
    plsc.store_scatter(acc1_v, [ivec], zerof + jnp.sum(a1), mask=lane0)
    return 0

  lax.fori_loop(0, BPW, batch_body, 0)

  dwv = dw_v[...]
  dw0 = dwv[0]
  dw1 = dwv[1]
  for t in range(BPW // 16):
    a0 = acc0_v[pl.ds(t * 16, 16)]
    a1 = acc1_v[pl.ds(t * 16, 16)]
    h0v = a0 * jnp.clip(a0 + 3.0, 0.0, 6.0) * (1.0 / 6.0)
    h1v = a1 * jnp.clip(a1 + 3.0, 0.0, 6.0) * (1.0 / 6.0)
    y = h0v * dw0 + h1v * dw1
    z = jnp.exp(-2.0 * jnp.abs(y))
    out_v[pl.ds(t * 16, 16)] = jnp.sign(y) * (1.0 - z) / (1.0 + z)

  pltpu.sync_copy(out_v, out_hbm.at[pl.ds(b0, BPW)])


@jax.jit
def kernel(inputs, table, conv_w, dense_w):
  idx = jnp.pad(inputs.astype(jnp.int32), ((0, 0), (0, LP - L)))
  sidx = lax.shift_right_logical(idx, 3).reshape(2 * B, HL)
  ridx = idx.reshape(2 * B, HL)
  tab0 = jnp.pad(table[:, 0], (0, PADP)).reshape(VSP, 8)
  tab1 = jnp.pad(table[:, 1], (0, PADP)).reshape(VSP, 8)
  w2 = jnp.pad(conv_w[:, 0, :].astype(jnp.float32), ((0, 0), (0, LP - L)))
  dw = jnp.pad(dense_w.reshape(2).astype(jnp.float32), (0, 14))

  mesh = plsc.VectorSubcoreMesh(core_axis_name="c", subcore_axis_name="s")
  out = pl.kernel(
      _sc_body,
      out_type=jax.ShapeDtypeStruct((B,), jnp.float32),
      mesh=mesh,
      compiler_params=pltpu.CompilerParams(
          needs_layout_passes=False, use_tc_tiling_on_sc=False),
      scratch_types=[
          pltpu.VMEM((HL,), jnp.int32),
          pltpu.VMEM((HL,), jnp.int32),
          pltpu.VMEM((HL,), jnp.int32),
          pltpu.VMEM((HL,), jnp.int32),
          pltpu.VMEM((HL, 8), jnp.float32),
          pltpu.VMEM((HL, 8), jnp.float32),
          pltpu.VMEM((HL, 8), jnp.float32),
          pltpu.VMEM((HL, 8), jnp.float32),
          pltpu.VMEM((LP,), jnp.float32),
          pltpu.VMEM((LP,), jnp.float32),
          pltpu.VMEM((16,), jnp.float32),
          pltpu.VMEM((BPW,), jnp.float32),
          pltpu.VMEM((BPW,), jnp.float32),
          pltpu.VMEM((BPW,), jnp.float32),
          pltpu.SemaphoreType.DMA,
          pltpu.SemaphoreType.DMA,
      ],
  )(sidx, ridx, tab0, tab1, w2, dw)
  return out.reshape(B, 1)
